# baseline (speedup 1.0000x reference)
import functools

import jax
import jax.numpy as jnp
from jax import lax
from jax.experimental import pallas as pl
from jax.experimental.pallas import tpu as pltpu
from jax.experimental.pallas import tpu_sc as plsc

N_VOCAB = 100000
EMBED_DIM = 64
PADDED_DIM = 128
OFFSET = 20
BATCH = 4096

_INFO = plsc.get_sparse_core_info()
NC = _INFO.num_cores
NS = _INFO.num_subcores
NW = NC * NS
B_PER_W = BATCH // NW
NB = 8
NSUB = B_PER_W // NB
ROWS = NB * OFFSET
W_IDX = B_PER_W * OFFSET

N_TILES = N_VOCAB // 8
CH = 20
N_CHUNKS = N_TILES // CH
CPW = -(-N_CHUNKS // NW)

_PARAMS = pltpu.CompilerParams(use_tc_tiling_on_sc=True)


def _make_depad():
    mesh = plsc.VectorSubcoreMesh(core_axis_name="c", subcore_axis_name="s")

    @functools.partial(
        pl.kernel,
        mesh=mesh,
        out_type=jax.ShapeDtypeStruct((N_VOCAB // 2, PADDED_DIM),
                                      jnp.float32),
        scratch_types=[
            pltpu.VMEM((2, CH, 8, EMBED_DIM), jnp.float32),
            pltpu.VMEM((2, CH * 4, PADDED_DIM), jnp.float32),
            pltpu.SemaphoreType.DMA((2,)),
            pltpu.SemaphoreType.DMA((2,)),
        ],
        compiler_params=_PARAMS,
    )
    def depad(w_hbm, out_hbm, tb, tb2, rsem, wsem):
        v3 = w_hbm.reshape(N_TILES, 8, EMBED_DIM)
        wid = lax.axis_index("s") * NC + lax.axis_index("c")

        def read(r):
            c = wid + NW * r
            return pltpu.async_copy(v3.at[pl.ds(c * CH, CH)],
                                    tb.at[r % 2], rsem.at[r % 2])

        def repack(r):
            buf = r % 2

            def body(q, _):
                i0 = q // 4
                j0 = 2 * (q % 4)
                for h in range(EMBED_DIM // 16):
                    sl = pl.ds(h * 16, 16)
                    tb2[buf, q, sl] = tb[buf, i0, j0, sl]
                    tb2[buf, q, pl.ds(EMBED_DIM + h * 16, 16)] = \
                        tb[buf, i0, j0 + 1, sl]
                return 0

            lax.fori_loop(0, CH * 4, body, 0)

        def write(r):
            c = wid + NW * r
            return pltpu.async_copy(
                tb2.at[r % 2],
                out_hbm.at[pl.ds(c * CH * 4, CH * 4)], wsem.at[r % 2])

        FULL = N_CHUNKS // NW
        rc = {0: read(0)}
        wc = {}
        for r in range(FULL):
            if r + 1 < FULL:
                rc[r + 1] = read(r + 1)
            rc[r].wait()
            if r >= 2:
                wc[r - 2].wait()
            repack(r)
            wc[r] = write(r)
        wc[FULL - 2].wait()
        wc[FULL - 1].wait()
        if N_CHUNKS % NW:
            @pl.when(wid + NW * FULL < N_CHUNKS)
            def _():
                read(FULL).wait()
                repack(FULL)
                write(FULL).wait()

    return depad


def _make_embag():
    mesh = plsc.VectorSubcoreMesh(core_axis_name="c", subcore_axis_name="s")

    @functools.partial(
        pl.kernel,
        mesh=mesh,
        out_type=jax.ShapeDtypeStruct((BATCH, PADDED_DIM), jnp.float32),
        scratch_types=[
            pltpu.VMEM((W_IDX,), jnp.int32),
            pltpu.VMEM((W_IDX,), jnp.int32),
            pltpu.VMEM((2, ROWS, PADDED_DIM), jnp.float32),
            pltpu.VMEM((2, NB, PADDED_DIM), jnp.float32),
            pltpu.SemaphoreType.DMA((2,)),
            pltpu.SemaphoreType.DMA((2,)),
        ],
        compiler_params=_PARAMS,
    )
    def embag(pidx_hbm, off_hbm, table_hbm, out_hbm,
              pidx_v, off_v, rows_v, out_v, gsem, osem):
        wid = lax.axis_index("s") * NC + lax.axis_index("c")
        wbase = wid * B_PER_W
        pltpu.sync_copy(pidx_hbm.at[pl.ds(wbase * OFFSET, W_IDX)], pidx_v)
        pltpu.sync_copy(off_hbm.at[pl.ds(wbase * OFFSET, W_IDX)], off_v)

        def gather(s):
            return pltpu.async_copy(
                table_hbm.at[pidx_v.at[pl.ds(s * ROWS, ROWS)]],
                rows_v.at[s % 2], gsem.at[s % 2])

        gc = {0: gather(0)}
        oc = {}
        for s in range(NSUB):
            if s + 1 < NSUB:
                gc[s + 1] = gather(s + 1)
            gc[s].wait()
            if s >= 2:
                oc[s - 2].wait()

            def body(b, _, buf=s % 2, s=s):
                ro = b * OFFSET
                o0 = off_v[pl.ds(s * ROWS + ro, 16)]
                o1 = off_v[pl.ds(s * ROWS + ro + 4, 16)]
                offs = [o0[j] if j < 16 else o1[j - 4] for j in range(OFFSET)]
                for v in range(EMBED_DIM // 16):
                    acc = rows_v[buf, ro, pl.ds(offs[0] + v * 16, 16)]
                    for j in range(1, OFFSET):
                        acc = acc + rows_v[buf, ro + j,
                                           pl.ds(offs[j] + v * 16, 16)]
                    out_v[buf, b, pl.ds(v * 16, 16)] = acc
                return 0

            lax.fori_loop(0, NB, body, 0)
            oc[s] = pltpu.async_copy(
                out_v.at[s % 2],
                out_hbm.at[pl.ds(wbase + s * NB, NB)], osem.at[s % 2])
        oc[NSUB - 2].wait()
        oc[NSUB - 1].wait()

    return embag


_depad = _make_depad()
_embag = _make_embag()


@jax.jit
def kernel(sparse_index_group_batch, sparse_offset_group_batch, weight):
    del sparse_offset_group_batch
    idx = sparse_index_group_batch.astype(jnp.int32)
    pidx = idx >> 1
    off = (idx & 1) * EMBED_DIM
    table = _depad(weight)
    return _embag(pidx, off, table)[:, :EMBED_DIM]


# speedup vs baseline: 1.1661x; 1.1661x over previous
import functools

import jax
import jax.numpy as jnp
from jax import lax
from jax.experimental import pallas as pl
from jax.experimental.pallas import tpu as pltpu
from jax.experimental.pallas import tpu_sc as plsc

N_VOCAB = 100000
EMBED_DIM = 64
PADDED_DIM = 128
OFFSET = 20
BATCH = 4096

_INFO = plsc.get_sparse_core_info()
NC = _INFO.num_cores
NS = _INFO.num_subcores
NW = NC * NS
B_PER_W = BATCH // NW
NB = 16
NSUB = B_PER_W // NB
ROWS = NB * OFFSET
W_IDX = B_PER_W * OFFSET

_PARAMS = pltpu.CompilerParams(use_tc_tiling_on_sc=True)


def _make_embag():
    mesh = plsc.VectorSubcoreMesh(core_axis_name="c", subcore_axis_name="s")

    @functools.partial(
        pl.kernel,
        mesh=mesh,
        out_type=jax.ShapeDtypeStruct((BATCH, PADDED_DIM), jnp.float32),
        scratch_types=[
            pltpu.VMEM((W_IDX,), jnp.int32),
            pltpu.VMEM((W_IDX,), jnp.int32),
            pltpu.VMEM((2, ROWS, PADDED_DIM), jnp.float32),
            pltpu.VMEM((2, NB, PADDED_DIM), jnp.float32),
            pltpu.SemaphoreType.DMA((2,)),
            pltpu.SemaphoreType.DMA((2,)),
        ],
        compiler_params=_PARAMS,
    )
    def embag(pidx_hbm, off_hbm, table_hbm, out_hbm,
              pidx_v, off_v, rows_v, out_v, gsem, osem):
        wid = lax.axis_index("s") * NC + lax.axis_index("c")
        wbase = wid * B_PER_W
        pltpu.sync_copy(pidx_hbm.at[pl.ds(wbase * OFFSET, W_IDX)], pidx_v)
        pltpu.sync_copy(off_hbm.at[pl.ds(wbase * OFFSET, W_IDX)], off_v)

        def gather(s):
            return pltpu.async_copy(
                table_hbm.at[pidx_v.at[pl.ds(s * ROWS, ROWS)]],
                rows_v.at[s % 2], gsem.at[s % 2])

        gc = {0: gather(0)}
        oc = {}
        for s in range(NSUB):
            if s + 1 < NSUB:
                gc[s + 1] = gather(s + 1)
            gc[s].wait()
            if s >= 2:
                oc[s - 2].wait()

            def body(b, _, buf=s % 2, s=s):
                ro = b * OFFSET
                o0 = off_v[pl.ds(s * ROWS + ro, 16)]
                o1 = off_v[pl.ds(s * ROWS + ro + 4, 16)]
                offs = [o0[j] if j < 16 else o1[j - 4] for j in range(OFFSET)]
                for v in range(EMBED_DIM // 16):
                    acc = rows_v[buf, ro, pl.ds(offs[0] + v * 16, 16)]
                    for j in range(1, OFFSET):
                        acc = acc + rows_v[buf, ro + j,
                                           pl.ds(offs[j] + v * 16, 16)]
                    out_v[buf, b, pl.ds(v * 16, 16)] = acc
                return 0

            lax.fori_loop(0, NB, body, 0)
            oc[s] = pltpu.async_copy(
                out_v.at[s % 2],
                out_hbm.at[pl.ds(wbase + s * NB, NB)], osem.at[s % 2])
        oc[NSUB - 2].wait()
        oc[NSUB - 1].wait()

    return embag


_embag = _make_embag()


@jax.jit
def kernel(sparse_index_group_batch, sparse_offset_group_batch, weight):
    del sparse_offset_group_batch
    idx = sparse_index_group_batch.astype(jnp.int32)
    pidx = idx >> 1
    off = (idx & 1) * EMBED_DIM
    one = lax.optimization_barrier(jnp.float32(1.0))
    table = (weight * one).reshape(N_VOCAB // 2, PADDED_DIM)
    return _embag(pidx, off, table)[:, :EMBED_DIM]


# speedup vs baseline: 1.3352x; 1.1450x over previous
import functools

import jax
import jax.numpy as jnp
from jax import lax
from jax.experimental import pallas as pl
from jax.experimental.pallas import tpu as pltpu
from jax.experimental.pallas import tpu_sc as plsc

N_VOCAB = 100000
EMBED_DIM = 64
PADDED_DIM = 128
OFFSET = 20
BATCH = 4096

_INFO = plsc.get_sparse_core_info()
NC = _INFO.num_cores
NS = _INFO.num_subcores
NW = NC * NS
B_PER_W = BATCH // NW
NB = 16
NSUB = B_PER_W // NB
ROWS = NB * OFFSET
W_IDX = B_PER_W * OFFSET

_PARAMS = pltpu.CompilerParams(use_tc_tiling_on_sc=False)


def _make_embag():
    mesh = plsc.VectorSubcoreMesh(core_axis_name="c", subcore_axis_name="s")

    @functools.partial(
        pl.kernel,
        mesh=mesh,
        out_type=jax.ShapeDtypeStruct((BATCH, PADDED_DIM), jnp.float32),
        scratch_types=[
            pltpu.VMEM((W_IDX,), jnp.int32),
            pltpu.VMEM((W_IDX,), jnp.int32),
            pltpu.VMEM((2, ROWS, PADDED_DIM), jnp.float32),
            pltpu.VMEM((2, NB, PADDED_DIM), jnp.float32),
            pltpu.SemaphoreType.DMA((2,)),
            pltpu.SemaphoreType.DMA((2,)),
        ],
        compiler_params=_PARAMS,
    )
    def embag(pidx_hbm, off_hbm, table_hbm, out_hbm,
              pidx_v, off_v, rows_v, out_v, gsem, osem):
        wid = lax.axis_index("s") * NC + lax.axis_index("c")
        wbase = wid * B_PER_W
        pltpu.sync_copy(pidx_hbm.at[pl.ds(wbase * OFFSET, W_IDX)], pidx_v)
        pltpu.sync_copy(off_hbm.at[pl.ds(wbase * OFFSET, W_IDX)], off_v)

        def gather(s):
            return pltpu.async_copy(
                table_hbm.at[pidx_v.at[pl.ds(s * ROWS, ROWS)]],
                rows_v.at[s % 2], gsem.at[s % 2])

        gc = {0: gather(0)}
        oc = {}
        for s in range(NSUB):
            if s + 1 < NSUB:
                gc[s + 1] = gather(s + 1)
            gc[s].wait()
            if s >= 2:
                oc[s - 2].wait()

            def body(b, _, buf=s % 2, s=s):
                ro = b * OFFSET
                o0 = off_v[pl.ds(s * ROWS + ro, 16)]
                o1 = off_v[pl.ds(s * ROWS + ro + 4, 16)]
                offs = [o0[j] if j < 16 else o1[j - 4] for j in range(OFFSET)]
                for v in range(EMBED_DIM // 16):
                    acc = rows_v[buf, ro, pl.ds(offs[0] + v * 16, 16)]
                    for j in range(1, OFFSET):
                        acc = acc + rows_v[buf, ro + j,
                                           pl.ds(offs[j] + v * 16, 16)]
                    out_v[buf, b, pl.ds(v * 16, 16)] = acc
                return 0

            lax.fori_loop(0, NB, body, 0)
            oc[s] = pltpu.async_copy(
                out_v.at[s % 2],
                out_hbm.at[pl.ds(wbase + s * NB, NB)], osem.at[s % 2])
        oc[NSUB - 2].wait()
        oc[NSUB - 1].wait()

    return embag


_embag = _make_embag()


@jax.jit
def kernel(sparse_index_group_batch, sparse_offset_group_batch, weight):
    del sparse_offset_group_batch
    idx = sparse_index_group_batch.astype(jnp.int32)
    pidx = idx >> 1
    off = (idx & 1) * EMBED_DIM
    table = weight.reshape(N_VOCAB // 2, PADDED_DIM)
    return _embag(pidx, off, table)[:, :EMBED_DIM]


# speedup vs baseline: 1.4410x; 1.0793x over previous
import functools

import jax
import jax.numpy as jnp
from jax import lax
from jax.experimental import pallas as pl
from jax.experimental.pallas import tpu as pltpu
from jax.experimental.pallas import tpu_sc as plsc

N_VOCAB = 100000
EMBED_DIM = 64
OFFSET = 20
BATCH = 4096

_INFO = plsc.get_sparse_core_info()
NC = _INFO.num_cores
NS = _INFO.num_subcores
NW = NC * NS
B_PER_W = BATCH // NW
NB = 32
NSUB = B_PER_W // NB
ROWS = NB * OFFSET


def _make_kernel():
    mesh = plsc.VectorSubcoreMesh(core_axis_name="c", subcore_axis_name="s")

    @functools.partial(
        pl.kernel,
        mesh=mesh,
        out_type=jax.ShapeDtypeStruct((BATCH, EMBED_DIM), jnp.float32),
        scratch_types=[
            pltpu.VMEM((B_PER_W * OFFSET,), jnp.int32),
            pltpu.VMEM((2, ROWS, EMBED_DIM), jnp.float32),
            pltpu.VMEM((2, NB, EMBED_DIM), jnp.float32),
            pltpu.SemaphoreType.DMA((2,)),
            pltpu.SemaphoreType.DMA((2,)),
        ],
        compiler_params=pltpu.CompilerParams(use_tc_tiling_on_sc=False),
    )
    def embag(idx_hbm, table_hbm, out_hbm, idx_v, rows_v, out_v, gsem, osem):
        wid = lax.axis_index("s") * NC + lax.axis_index("c")
        wbase = wid * B_PER_W
        pltpu.sync_copy(idx_hbm.at[pl.ds(wbase * OFFSET, B_PER_W * OFFSET)],
                        idx_v)

        def gather(s):
            return pltpu.async_copy(
                table_hbm.at[idx_v.at[pl.ds(s * ROWS, ROWS)]],
                rows_v.at[s % 2], gsem.at[s % 2])

        gc = {0: gather(0)}
        oc = {}
        for s in range(NSUB):
            if s + 1 < NSUB:
                gc[s + 1] = gather(s + 1)
            gc[s].wait()
            if s >= 2:
                oc[s - 2].wait()

            def body(b, _, buf=s % 2):
                for v in range(EMBED_DIM // 16):
                    sl = pl.ds(v * 16, 16)
                    acc = rows_v[buf, b * OFFSET, sl]
                    for j in range(1, OFFSET):
                        acc = acc + rows_v[buf, b * OFFSET + j, sl]
                    out_v[buf, b, sl] = acc
                return 0

            lax.fori_loop(0, NB, body, 0)
            oc[s] = pltpu.async_copy(
                out_v.at[s % 2],
                out_hbm.at[pl.ds(wbase + s * NB, NB)], osem.at[s % 2])
        oc[NSUB - 2].wait()
        oc[NSUB - 1].wait()

    return embag


_embag = _make_kernel()


@jax.jit
def kernel(sparse_index_group_batch, sparse_offset_group_batch, weight):
    del sparse_offset_group_batch
    idx = sparse_index_group_batch.astype(jnp.int32)
    return _embag(idx, weight)
